# baseline (speedup 1.0000x reference)
import functools

import jax
import jax.numpy as jnp
from jax import lax
from jax.experimental import pallas as pl
from jax.experimental.pallas import tpu as pltpu
from jax.experimental.pallas import tpu_sc as plsc

_D = 128
_B = 16384

_NC = 2
_NS = 8
_NW = _NC * _NS
_BPW = _B // _NW
_CH = 256
_NCH = _BPW // _CH

_mesh = plsc.VectorSubcoreMesh(
    core_axis_name="c", subcore_axis_name="s", num_cores=_NC, num_subcores=_NS)


@functools.partial(
    pl.kernel,
    mesh=_mesh,
    out_type=jax.ShapeDtypeStruct((_B, _D), jnp.float32),
    scratch_types=[
        pltpu.VMEM((_BPW,), jnp.int32),
        pltpu.VMEM((2, _CH, _D), jnp.float32),
        pltpu.SemaphoreType.DMA,
        pltpu.SemaphoreType.DMA,
        pltpu.SemaphoreType.DMA,
        pltpu.SemaphoreType.DMA,
    ],
)
def _emb_gather(idx_hbm, table_hbm, out_hbm, idx_v, bufs, g0, g1, s0, s1):
    wid = lax.axis_index("s") * _NC + lax.axis_index("c")
    base = wid * _BPW
    pltpu.sync_copy(idx_hbm.at[pl.ds(base, _BPW)], idx_v)
    gsems = (g0, g1)
    ssems = (s0, s1)
    gathers = [None, None]
    stores = [None, None]
    for k in range(_NCH):
        b = k % 2
        if stores[b] is not None:
            stores[b].wait()
        gathers[b] = pltpu.async_copy(
            table_hbm.at[idx_v.at[pl.ds(k * _CH, _CH)]], bufs.at[b], gsems[b])
        gathers[b].wait()
        stores[b] = pltpu.async_copy(
            bufs.at[b], out_hbm.at[pl.ds(base + k * _CH, _CH)], ssems[b])
    stores[0].wait()
    stores[1].wait()


def kernel(titles, embedding_table):
    return _emb_gather(titles.astype(jnp.int32), embedding_table)


# speedup vs baseline: 1.2006x; 1.2006x over previous
import functools

import jax
import jax.numpy as jnp
from jax import lax
from jax.experimental import pallas as pl
from jax.experimental.pallas import tpu as pltpu
from jax.experimental.pallas import tpu_sc as plsc

_D = 128
_B = 16384

_info = plsc.get_sparse_core_info()
_NC = _info.num_cores
_NS = _info.num_subcores
_NW = _NC * _NS
_BPW = _B // _NW

_mesh = plsc.VectorSubcoreMesh(core_axis_name="c", subcore_axis_name="s")


@functools.partial(
    pl.kernel,
    mesh=_mesh,
    out_type=jax.ShapeDtypeStruct((_B, _D), jnp.float32),
    scratch_types=[
        pltpu.VMEM((_BPW,), jnp.int32),
        pltpu.VMEM((_BPW, _D), jnp.float32),
        pltpu.SemaphoreType.DMA,
        pltpu.SemaphoreType.DMA,
        pltpu.SemaphoreType.DMA,
    ],
)
def _emb_gather(idx_hbm, table_hbm, out_hbm, idx_v, rows_v, g0, g1, s_sem):
    wid = lax.axis_index("s") * _NC + lax.axis_index("c")
    base = wid * _BPW
    half = _BPW // 2
    pltpu.sync_copy(idx_hbm.at[pl.ds(base, _BPW)], idx_v)
    ga = pltpu.async_copy(
        table_hbm.at[idx_v.at[pl.ds(0, half)]], rows_v.at[pl.ds(0, half)], g0)
    gb = pltpu.async_copy(
        table_hbm.at[idx_v.at[pl.ds(half, half)]], rows_v.at[pl.ds(half, half)], g1)
    ga.wait()
    sa = pltpu.async_copy(
        rows_v.at[pl.ds(0, half)], out_hbm.at[pl.ds(base, half)], s_sem)
    gb.wait()
    sb = pltpu.async_copy(
        rows_v.at[pl.ds(half, half)], out_hbm.at[pl.ds(base + half, half)], s_sem)
    sa.wait()
    sb.wait()


def kernel(titles, embedding_table):
    return _emb_gather(titles.astype(jnp.int32), embedding_table)
